# Initial kernel scaffold; baseline (speedup 1.0000x reference)
import jax
import jax.numpy as jnp
from jax.experimental import pallas as pl


def kernel(noise_bursts, segment_ids, logits):
    raise NotImplementedError("write your pallas kernel here")


# speedup vs baseline: 233.0009x; 233.0009x over previous
import functools
import math

import jax
import jax.numpy as jnp
from jax.experimental import pallas as pl
from jax.experimental.pallas import tpu as pltpu

GAIN_MIN = 0.1
GAIN_MAX = 2.0
SR = 16000
LOG_MIN_W = math.log(2.0 * math.pi * 20.0 / SR)
LOG_MAX_W = math.log(math.pi)
LOG_MIN_Q = math.log(0.0707)
LOG_MAX_Q = math.log(2.0)

NSEG = 64
CHUNK_L = 64
CHUNK_K = 64


def _coef_kernel(seg_ref, logits_ref, noise_ref, f_ref, a1_ref, a2_ref):
    ids = seg_ref[0]
    logits = logits_ref[0]
    T = ids.shape[1]

    iota_s = jax.lax.broadcasted_iota(jnp.int32, (NSEG, T), 0)
    mask = (iota_s == ids).astype(jnp.float32)
    sums = jax.lax.dot_general(mask, logits, (((1,), (1,)), ((), ())),
                               preferred_element_type=jnp.float32)
    counts = jnp.sum(mask, axis=1, keepdims=True)
    means = sums / jnp.maximum(counts, 1.0)
    seg_c = jax.lax.dot_general(means, mask, (((0,), (0,)), ((), ())),
                                preferred_element_type=jnp.float32)

    gain = GAIN_MIN + (GAIN_MAX - GAIN_MIN) * jax.nn.sigmoid(seg_c[0:1, :])
    w = jnp.exp(LOG_MIN_W + jax.nn.sigmoid(seg_c[1:2, :]) * (LOG_MAX_W - LOG_MIN_W))
    q = jnp.exp(LOG_MIN_Q + jax.nn.sigmoid(seg_c[2:3, :]) * (LOG_MAX_Q - LOG_MIN_Q))
    cosw = jnp.cos(w)
    alpha = jnp.sin(w) / (2.0 * q)
    inv_a0 = 1.0 / (1.0 + alpha)
    omc = 1.0 - cosw
    b0 = 0.5 * omc * inv_a0
    b1 = omc * inv_a0

    x = noise_ref[0] * gain
    z1 = jnp.zeros((1, 1), jnp.float32)
    x1 = jnp.concatenate([z1, x[:, :-1]], axis=1)
    x2 = jnp.concatenate([z1, z1, x[:, :-2]], axis=1)
    f_ref[0] = b0 * x + b1 * x1 + b0 * x2
    a1_ref[0] = -2.0 * cosw * inv_a0
    a2_ref[0] = (1.0 - alpha) * inv_a0


def _scan_kernel(f_ref, a1_ref, a2_ref, y_ref, u_ref, v_ref, d_ref, *, n_rows):
    L = f_ref.shape[0]
    KB = f_ref.shape[1]
    K = KB // n_rows

    ones = jnp.ones((1, KB), jnp.float32)
    zeros = jnp.zeros((1, KB), jnp.float32)
    u1, u2, v1, v2, d1, d2 = ones, zeros, zeros, ones, zeros, zeros
    for l in range(L):
        a1 = a1_ref[l:l + 1, :]
        a2 = a2_ref[l:l + 1, :]
        fl = f_ref[l:l + 1, :]
        u = -a1 * u1 - a2 * u2
        v = -a1 * v1 - a2 * v2
        d = fl - a1 * d1 - a2 * d2
        u_ref[l:l + 1, :] = u
        v_ref[l:l + 1, :] = v
        d_ref[l:l + 1, :] = d
        u1, u2, v1, v2, d1, d2 = u, u1, v, v1, d, d1

    u_last = u_ref[L - 1:L, :]
    u_prev = u_ref[L - 2:L - 1, :]
    v_last = v_ref[L - 1:L, :]
    v_prev = v_ref[L - 2:L - 1, :]
    d_last = d_ref[L - 1:L, :]
    d_prev = d_ref[L - 2:L - 1, :]

    y1 = jnp.zeros((1, n_rows), jnp.float32)
    y2 = jnp.zeros((1, n_rows), jnp.float32)
    y1_parts = []
    y2_parts = []
    for k in range(K):
        sl = slice(k * n_rows, (k + 1) * n_rows)
        y1_parts.append(y1)
        y2_parts.append(y2)
        y_end = u_last[:, sl] * y1 + v_last[:, sl] * y2 + d_last[:, sl]
        y_end2 = u_prev[:, sl] * y1 + v_prev[:, sl] * y2 + d_prev[:, sl]
        y1, y2 = y_end, y_end2
    y1_all = jnp.concatenate(y1_parts, axis=1)
    y2_all = jnp.concatenate(y2_parts, axis=1)

    y_ref[:, :] = u_ref[:, :] * y1_all + v_ref[:, :] * y2_all + d_ref[:, :]


def kernel(noise_bursts, segment_ids, logits):
    B, T = noise_bursts.shape
    L, K = CHUNK_L, CHUNK_K
    seg = segment_ids.astype(jnp.int32).reshape(B, 1, T)
    logits_t = jnp.transpose(logits, (0, 2, 1))
    noise3 = noise_bursts.reshape(B, 1, T)

    f, a1, a2 = pl.pallas_call(
        _coef_kernel,
        grid=(B,),
        in_specs=[
            pl.BlockSpec((1, 1, T), lambda b: (b, 0, 0)),
            pl.BlockSpec((1, 3, T), lambda b: (b, 0, 0)),
            pl.BlockSpec((1, 1, T), lambda b: (b, 0, 0)),
        ],
        out_specs=[
            pl.BlockSpec((1, 1, T), lambda b: (b, 0, 0)),
            pl.BlockSpec((1, 1, T), lambda b: (b, 0, 0)),
            pl.BlockSpec((1, 1, T), lambda b: (b, 0, 0)),
        ],
        out_shape=[jax.ShapeDtypeStruct((B, 1, T), jnp.float32)] * 3,
    )(seg, logits_t, noise3)

    def to_scan(arr):
        return arr.reshape(B, K, L).transpose(2, 1, 0).reshape(L, K * B)

    y2d = pl.pallas_call(
        functools.partial(_scan_kernel, n_rows=B),
        out_shape=jax.ShapeDtypeStruct((L, K * B), jnp.float32),
        scratch_shapes=[pltpu.VMEM((L, K * B), jnp.float32)] * 3,
    )(to_scan(f), to_scan(a1), to_scan(a2))

    return y2d.reshape(L, K, B).transpose(2, 1, 0).reshape(B, T)


# speedup vs baseline: 605.5417x; 2.5989x over previous
import math

import jax
import jax.numpy as jnp
from jax.experimental import pallas as pl
from jax.experimental.pallas import tpu as pltpu

GAIN_MIN = 0.1
GAIN_MAX = 2.0
SR = 16000
LOG_MIN_W = math.log(2.0 * math.pi * 20.0 / SR)
LOG_MAX_W = math.log(math.pi)
LOG_MIN_Q = math.log(0.0707)
LOG_MAX_Q = math.log(2.0)

NSEG = 64
CHUNK_L = 128
CHUNK_K = 32


def _fused_kernel(seg_ref, noise_ref, logits_ref, y_ref,
                  g0_ref, g1_ref, g2_ref, sf_ref, sa1_ref, sa2_ref,
                  su_ref, sv_ref, sd_ref):
    B, T = seg_ref.shape
    L, K = CHUNK_L, CHUNK_K
    KB = B * K

    ones_t = jnp.ones((1, T), jnp.float32)
    for b in range(B):
        ids = seg_ref[b:b + 1, :]
        lg = logits_ref[:, b, :]
        iota_s = jax.lax.broadcasted_iota(jnp.int32, (NSEG, T), 0)
        mask = (iota_s == ids).astype(jnp.float32)
        sums = jax.lax.dot_general(mask, lg, (((1,), (1,)), ((), ())),
                                   preferred_element_type=jnp.float32)
        counts = jax.lax.dot_general(mask, ones_t, (((1,), (1,)), ((), ())),
                                     preferred_element_type=jnp.float32)
        means = sums / jnp.maximum(counts, 1.0)
        seg_c = jax.lax.dot_general(means, mask, (((0,), (0,)), ((), ())),
                                    preferred_element_type=jnp.float32)
        g0_ref[b:b + 1, :] = seg_c[0:1, :]
        g1_ref[b:b + 1, :] = seg_c[1:2, :]
        g2_ref[b:b + 1, :] = seg_c[2:3, :]

    gain = GAIN_MIN + (GAIN_MAX - GAIN_MIN) * jax.nn.sigmoid(g0_ref[:, :])
    w = jnp.exp(LOG_MIN_W + jax.nn.sigmoid(g1_ref[:, :]) * (LOG_MAX_W - LOG_MIN_W))
    q = jnp.exp(LOG_MIN_Q + jax.nn.sigmoid(g2_ref[:, :]) * (LOG_MAX_Q - LOG_MIN_Q))
    cosw = jnp.cos(w)
    alpha = jnp.sin(w) / (2.0 * q)
    inv_a0 = 1.0 / (1.0 + alpha)
    omc = 1.0 - cosw
    b0 = 0.5 * omc * inv_a0
    b1 = omc * inv_a0
    a1c = -2.0 * cosw * inv_a0
    a2c = (1.0 - alpha) * inv_a0

    x = noise_ref[:, :] * gain
    zc = jnp.zeros((B, 1), jnp.float32)
    x1 = jnp.concatenate([zc, x[:, :-1]], axis=1)
    x2 = jnp.concatenate([zc, zc, x[:, :-2]], axis=1)
    fv = b0 * x + b1 * x1 + b0 * x2

    for b in range(B):
        cs = slice(b * K, (b + 1) * K)
        sf_ref[:, cs] = jnp.transpose(fv[b:b + 1, :].reshape(K, L))
        sa1_ref[:, cs] = jnp.transpose(a1c[b:b + 1, :].reshape(K, L))
        sa2_ref[:, cs] = jnp.transpose(a2c[b:b + 1, :].reshape(K, L))

    ones = jnp.ones((1, KB), jnp.float32)
    zeros = jnp.zeros((1, KB), jnp.float32)
    u1, u2, v1, v2, d1, d2 = ones, zeros, zeros, ones, zeros, zeros
    for l in range(L):
        a1 = sa1_ref[l:l + 1, :]
        a2 = sa2_ref[l:l + 1, :]
        fl = sf_ref[l:l + 1, :]
        u = -a1 * u1 - a2 * u2
        v = -a1 * v1 - a2 * v2
        d = fl - a1 * d1 - a2 * d2
        su_ref[l:l + 1, :] = u
        sv_ref[l:l + 1, :] = v
        sd_ref[l:l + 1, :] = d
        u1, u2, v1, v2, d1, d2 = u, u1, v, v1, d, d1

    m00 = su_ref[L - 1:L, :]
    m01 = sv_ref[L - 1:L, :]
    m10 = su_ref[L - 2:L - 1, :]
    m11 = sv_ref[L - 2:L - 1, :]
    q0 = sd_ref[L - 1:L, :]
    q1 = sd_ref[L - 2:L - 1, :]

    kidx = jax.lax.rem(jax.lax.broadcasted_iota(jnp.int32, (1, KB), 1),
                       jnp.int32(K))

    def shift_k(arr, d, fill):
        pad = jnp.full((1, d), fill, jnp.float32)
        rolled = jnp.concatenate([pad, arr[:, :-d]], axis=1)
        return jnp.where(kidx >= d, rolled, fill)

    d = 1
    while d < K:
        s00 = shift_k(m00, d, 1.0)
        s01 = shift_k(m01, d, 0.0)
        s10 = shift_k(m10, d, 0.0)
        s11 = shift_k(m11, d, 1.0)
        t0 = shift_k(q0, d, 0.0)
        t1 = shift_k(q1, d, 0.0)
        n00 = m00 * s00 + m01 * s10
        n01 = m00 * s01 + m01 * s11
        n10 = m10 * s00 + m11 * s10
        n11 = m10 * s01 + m11 * s11
        nq0 = m00 * t0 + m01 * t1 + q0
        nq1 = m10 * t0 + m11 * t1 + q1
        m00, m01, m10, m11, q0, q1 = n00, n01, n10, n11, nq0, nq1
        d *= 2

    y1_all = shift_k(q0, 1, 0.0)
    y2_all = shift_k(q1, 1, 0.0)

    y = su_ref[:, :] * y1_all + sv_ref[:, :] * y2_all + sd_ref[:, :]
    for b in range(B):
        yb = jnp.transpose(y[:, b * K:(b + 1) * K])
        y_ref[b:b + 1, :] = yb.reshape(1, T)


def kernel(noise_bursts, segment_ids, logits):
    B, T = noise_bursts.shape
    seg = segment_ids.astype(jnp.int32)
    logits_t = jnp.transpose(logits, (2, 0, 1))

    return pl.pallas_call(
        _fused_kernel,
        out_shape=jax.ShapeDtypeStruct((B, T), jnp.float32),
        scratch_shapes=[pltpu.VMEM((B, T), jnp.float32)] * 3
        + [pltpu.VMEM((CHUNK_L, B * CHUNK_K), jnp.float32)] * 6,
    )(seg, noise_bursts, logits_t)


# speedup vs baseline: 631.7762x; 1.0433x over previous
import math

import jax
import jax.numpy as jnp
from jax.experimental import pallas as pl
from jax.experimental.pallas import tpu as pltpu

GAIN_MIN = 0.1
GAIN_MAX = 2.0
SR = 16000
LOG_MIN_W = math.log(2.0 * math.pi * 20.0 / SR)
LOG_MAX_W = math.log(math.pi)
LOG_MIN_Q = math.log(0.0707)
LOG_MAX_Q = math.log(2.0)

NSEG = 64
CHUNK_L = 128
CHUNK_K = 32


def _fused_kernel(seg_ref, noise_ref, logits_ref, y_ref,
                  pg_ref, pb0_ref, pb1_ref, pa1_ref, pa2_ref,
                  sf_ref, sa1_ref, sa2_ref, su_ref, sv_ref, sd_ref):
    B, T = seg_ref.shape
    L, K = CHUNK_L, CHUNK_K
    KB = B * K

    ones_t = jnp.ones((1, T), jnp.float32)
    iota_s = jax.lax.broadcasted_iota(jnp.int32, (NSEG, T), 0)
    for b in range(B):
        ids = seg_ref[b:b + 1, :]
        lg4 = jnp.concatenate([logits_ref[:, b, :], ones_t], axis=0)
        mask = (iota_s == ids).astype(jnp.float32)
        sc = jax.lax.dot_general(lg4, mask, (((1,), (1,)), ((), ())),
                                 preferred_element_type=jnp.float32)
        means = sc[0:3, :] / jnp.maximum(sc[3:4, :], 1.0)

        gain = GAIN_MIN + (GAIN_MAX - GAIN_MIN) * jax.nn.sigmoid(means[0:1, :])
        w = jnp.exp(LOG_MIN_W + jax.nn.sigmoid(means[1:2, :]) * (LOG_MAX_W - LOG_MIN_W))
        q = jnp.exp(LOG_MIN_Q + jax.nn.sigmoid(means[2:3, :]) * (LOG_MAX_Q - LOG_MIN_Q))
        cosw = jnp.cos(w)
        alpha = jnp.sin(w) / (2.0 * q)
        inv_a0 = 1.0 / (1.0 + alpha)
        omc = 1.0 - cosw
        b0 = 0.5 * omc * inv_a0
        b1 = omc * inv_a0
        a1 = -2.0 * cosw * inv_a0
        a2 = (1.0 - alpha) * inv_a0
        coef = jnp.concatenate([gain, b0, b1, a1, a2], axis=0)

        plane = jax.lax.dot_general(coef, mask, (((1,), (0,)), ((), ())),
                                    preferred_element_type=jnp.float32)
        pg_ref[b:b + 1, :] = plane[0:1, :]
        pb0_ref[b:b + 1, :] = plane[1:2, :]
        pb1_ref[b:b + 1, :] = plane[2:3, :]
        pa1_ref[b:b + 1, :] = plane[3:4, :]
        pa2_ref[b:b + 1, :] = plane[4:5, :]

    x = noise_ref[:, :] * pg_ref[:, :]
    zc = jnp.zeros((B, 1), jnp.float32)
    x1 = jnp.concatenate([zc, x[:, :-1]], axis=1)
    x2 = jnp.concatenate([zc, zc, x[:, :-2]], axis=1)
    fv = pb0_ref[:, :] * (x + x2) + pb1_ref[:, :] * x1
    a1c = pa1_ref[:, :]
    a2c = pa2_ref[:, :]

    for b in range(B):
        cs = slice(b * K, (b + 1) * K)
        sf_ref[:, cs] = jnp.transpose(fv[b:b + 1, :].reshape(K, L))
        sa1_ref[:, cs] = jnp.transpose(a1c[b:b + 1, :].reshape(K, L))
        sa2_ref[:, cs] = jnp.transpose(a2c[b:b + 1, :].reshape(K, L))

    ones = jnp.ones((1, KB), jnp.float32)
    zeros = jnp.zeros((1, KB), jnp.float32)
    u1, u2, v1, v2, d1, d2 = ones, zeros, zeros, ones, zeros, zeros
    for l in range(L):
        a1 = sa1_ref[l:l + 1, :]
        a2 = sa2_ref[l:l + 1, :]
        fl = sf_ref[l:l + 1, :]
        u = -a1 * u1 - a2 * u2
        v = -a1 * v1 - a2 * v2
        d = fl - a1 * d1 - a2 * d2
        su_ref[l:l + 1, :] = u
        sv_ref[l:l + 1, :] = v
        sd_ref[l:l + 1, :] = d
        u1, u2, v1, v2, d1, d2 = u, u1, v, v1, d, d1

    m00 = su_ref[L - 1:L, :]
    m01 = sv_ref[L - 1:L, :]
    m10 = su_ref[L - 2:L - 1, :]
    m11 = sv_ref[L - 2:L - 1, :]
    q0 = sd_ref[L - 1:L, :]
    q1 = sd_ref[L - 2:L - 1, :]

    kidx = jax.lax.rem(jax.lax.broadcasted_iota(jnp.int32, (1, KB), 1),
                       jnp.int32(K))

    def shift_k(arr, d, fill):
        pad = jnp.full((1, d), fill, jnp.float32)
        rolled = jnp.concatenate([pad, arr[:, :-d]], axis=1)
        return jnp.where(kidx >= d, rolled, fill)

    d = 1
    while d < K:
        s00 = shift_k(m00, d, 1.0)
        s01 = shift_k(m01, d, 0.0)
        s10 = shift_k(m10, d, 0.0)
        s11 = shift_k(m11, d, 1.0)
        t0 = shift_k(q0, d, 0.0)
        t1 = shift_k(q1, d, 0.0)
        n00 = m00 * s00 + m01 * s10
        n01 = m00 * s01 + m01 * s11
        n10 = m10 * s00 + m11 * s10
        n11 = m10 * s01 + m11 * s11
        nq0 = m00 * t0 + m01 * t1 + q0
        nq1 = m10 * t0 + m11 * t1 + q1
        m00, m01, m10, m11, q0, q1 = n00, n01, n10, n11, nq0, nq1
        d *= 2

    y1_all = shift_k(q0, 1, 0.0)
    y2_all = shift_k(q1, 1, 0.0)

    y = su_ref[:, :] * y1_all + sv_ref[:, :] * y2_all + sd_ref[:, :]
    for b in range(B):
        yb = jnp.transpose(y[:, b * K:(b + 1) * K])
        y_ref[b:b + 1, :] = yb.reshape(1, T)


def kernel(noise_bursts, segment_ids, logits):
    B, T = noise_bursts.shape
    seg = segment_ids.astype(jnp.int32)
    logits_t = jnp.transpose(logits, (2, 0, 1))

    return pl.pallas_call(
        _fused_kernel,
        out_shape=jax.ShapeDtypeStruct((B, T), jnp.float32),
        scratch_shapes=[pltpu.VMEM((B, T), jnp.float32)] * 5
        + [pltpu.VMEM((CHUNK_L, B * CHUNK_K), jnp.float32)] * 6,
    )(seg, noise_bursts, logits_t)
